# baseline (speedup 1.0000x reference)
import functools

import jax
import jax.numpy as jnp
from jax import lax
from jax.experimental import pallas as pl
from jax.experimental.pallas import tpu as pltpu

_RESIDENT_MAX = 2048

_K = 3

_C_BIG = 3

_DIMS = (((1,), (1,)), ((), ()))


def _dott(lhs, rhs):
    return lax.dot_general(lhs, rhs, _DIMS, preferred_element_type=jnp.float32)


def _prop_body(a2_ref, a1_ref, ht_ref, out_ref, buf, sem, a2_sc, a1_sc,
               t1, e1, t2, *, bm, grid_rows, c_strips):
    p = pl.program_id(0)
    i = pl.program_id(1)
    rows = pl.ds(i * bm, bm)
    g = grid_rows
    c = c_strips

    def issue(pos, slot):
        @pl.when(pos < g)
        def _from_a2():
            pltpu.make_async_copy(
                a2_ref.at[pl.ds(pos * bm, bm), :], buf.at[slot], sem.at[slot]
            ).start()

        @pl.when(pos >= g)
        def _from_a1():
            strip = jnp.where(pos < 2 * g, pos - g, pos - 2 * g + c)
            pltpu.make_async_copy(
                a1_ref.at[pl.ds(strip * bm, bm), :], buf.at[slot], sem.at[slot]
            ).start()

    n_pos = 3 * g - c
    consuming = (p == 0) | (p == 1) | ((p == 3) & (i >= c))
    pos = jnp.where(p == 0, i, jnp.where(p == 1, g + i, 2 * g + i - c))
    slot = lax.rem(pos, _K)

    @pl.when((p == 0) & (i == 0))
    def _prologue():
        for k in range(_K):
            issue(jnp.int32(k), jnp.int32(k))

    def wait_strip():
        pltpu.make_async_copy(
            a2_ref.at[pl.ds(0, bm), :], buf.at[slot], sem.at[slot]
        ).wait()

    def refill():
        nxt = pos + _K

        @pl.when(consuming & (nxt < n_pos))
        def _():
            issue(nxt, slot)

    @pl.when(p == 0)
    def _phase0():
        wait_strip()
        blk = buf[slot].astype(jnp.bfloat16)
        a2_sc[rows, :] = blk
        t1[:, rows] = _dott(ht_ref[...].astype(jnp.bfloat16), blk)

    @pl.when(p == 1)
    def _phase1():
        wait_strip()
        blk = buf[slot].astype(jnp.bfloat16)

        @pl.when(i < c)
        def _cache():
            a1_sc[rows, :] = blk

        e1[:, rows] = _dott(t1[...].astype(jnp.bfloat16), blk)

    @pl.when(p == 2)
    def _phase2():
        t2[:, rows] = _dott(e1[...].astype(jnp.bfloat16), a2_sc[rows, :])

    @pl.when(p == 3)
    def _phase3():
        t2_bf = t2[...].astype(jnp.bfloat16)
        base = ht_ref[:, rows] + e1[:, rows]

        @pl.when(i < c)
        def _from_cache():
            e2_blk = _dott(t2_bf, a1_sc[rows, :])
            out_ref[...] = (base + e2_blk) * (1.0 / 3.0)

        @pl.when(i >= c)
        def _from_stream():
            wait_strip()
            e2_blk = _dott(t2_bf, buf[slot].astype(jnp.bfloat16))
            out_ref[...] = (base + e2_blk) * (1.0 / 3.0)

    refill()


def _prop(a1, a2, h, *, bm):
    n, d = h.shape
    grid_rows = n // bm
    c_strips = grid_rows if n <= _RESIDENT_MAX else _C_BIG

    def out_map(p, i):
        return (0, jnp.where(p == 3, i, 0))

    scratch = [
        pltpu.VMEM((_K, bm, n), jnp.float32),
        pltpu.SemaphoreType.DMA((_K,)),
        pltpu.VMEM((n, n), jnp.bfloat16),
        pltpu.VMEM((c_strips * bm, n), jnp.bfloat16),
        pltpu.VMEM((d, n), jnp.float32),
        pltpu.VMEM((d, n), jnp.float32),
        pltpu.VMEM((d, n), jnp.float32),
    ]

    ht = h.T
    outt = pl.pallas_call(
        functools.partial(_prop_body, bm=bm, grid_rows=grid_rows,
                          c_strips=c_strips),
        grid=(4, grid_rows),
        in_specs=[
            pl.BlockSpec(memory_space=pl.ANY),
            pl.BlockSpec(memory_space=pl.ANY),
            pl.BlockSpec((d, n), lambda p, i: (0, 0)),
        ],
        out_specs=pl.BlockSpec((d, bm), out_map),
        out_shape=jax.ShapeDtypeStruct((d, n), jnp.float32),
        scratch_shapes=scratch,
        compiler_params=pltpu.CompilerParams(
            dimension_semantics=("arbitrary", "arbitrary"),
        ),
    )(a2, a1, ht)
    return outt.T


def kernel(adj_u1, adj_u2, adj_i1, adj_i2, adj_m1, adj_m2, adj_a1, adj_a2,
           user_emb, item_emb, mashup_tag_emb, api_tag_emb):
    u = _prop(adj_u1, adj_u2, user_emb, bm=256)
    i = _prop(adj_i1, adj_i2, item_emb, bm=256)
    m = _prop(adj_m1, adj_m2, mashup_tag_emb, bm=256)
    a = _prop(adj_a1, adj_a2, api_tag_emb, bm=256)
    return (u, i, m, a)


# speedup vs baseline: 1.0099x; 1.0099x over previous
import functools

import jax
import jax.numpy as jnp
from jax import lax
from jax.experimental import pallas as pl
from jax.experimental.pallas import tpu as pltpu

_RESIDENT_MAX = 2048

_K = 4

_C_BIG = 0

_DIMS = (((1,), (1,)), ((), ()))


def _dott(lhs, rhs):
    return lax.dot_general(lhs, rhs, _DIMS, preferred_element_type=jnp.float32)


def _prop_body(a2_ref, a1_ref, ht_ref, out_ref, buf, sem, a2_sc, a1_sc,
               t1, e1, t2, *, bm, grid_rows, c_strips):
    p = pl.program_id(0)
    i = pl.program_id(1)
    rows = pl.ds(i * bm, bm)
    g = grid_rows
    c = c_strips

    def issue(pos, slot):
        @pl.when(pos < g)
        def _from_a2():
            pltpu.make_async_copy(
                a2_ref.at[pl.ds(pos * bm, bm), :], buf.at[slot], sem.at[slot]
            ).start()

        @pl.when(pos >= g)
        def _from_a1():
            strip = jnp.where(pos < 2 * g, pos - g, pos - 2 * g + c)
            pltpu.make_async_copy(
                a1_ref.at[pl.ds(strip * bm, bm), :], buf.at[slot], sem.at[slot]
            ).start()

    n_pos = 3 * g - c
    consuming = (p == 0) | (p == 1) | ((p == 3) & (i >= c))
    pos = jnp.where(p == 0, i, jnp.where(p == 1, g + i, 2 * g + i - c))
    slot = lax.rem(pos, _K)

    @pl.when((p == 0) & (i == 0))
    def _prologue():
        for k in range(_K):
            issue(jnp.int32(k), jnp.int32(k))

    def wait_strip():
        pltpu.make_async_copy(
            a2_ref.at[pl.ds(0, bm), :], buf.at[slot], sem.at[slot]
        ).wait()

    def refill():
        nxt = pos + _K

        @pl.when(consuming & (nxt < n_pos))
        def _():
            issue(nxt, slot)

    @pl.when(p == 0)
    def _phase0():
        wait_strip()
        blk = buf[slot].astype(jnp.bfloat16)
        a2_sc[rows, :] = blk
        t1[:, rows] = _dott(ht_ref[...].astype(jnp.bfloat16), blk)

    @pl.when(p == 1)
    def _phase1():
        wait_strip()
        blk = buf[slot].astype(jnp.bfloat16)

        @pl.when(i < c)
        def _cache():
            a1_sc[rows, :] = blk

        e1[:, rows] = _dott(t1[...].astype(jnp.bfloat16), blk)

    @pl.when(p == 2)
    def _phase2():
        t2[:, rows] = _dott(e1[...].astype(jnp.bfloat16), a2_sc[rows, :])

    @pl.when(p == 3)
    def _phase3():
        t2_bf = t2[...].astype(jnp.bfloat16)
        base = ht_ref[:, rows] + e1[:, rows]

        @pl.when(i < c)
        def _from_cache():
            e2_blk = _dott(t2_bf, a1_sc[rows, :])
            out_ref[...] = (base + e2_blk) * (1.0 / 3.0)

        @pl.when(i >= c)
        def _from_stream():
            wait_strip()
            e2_blk = _dott(t2_bf, buf[slot].astype(jnp.bfloat16))
            out_ref[...] = (base + e2_blk) * (1.0 / 3.0)

    refill()


def _prop(a1, a2, h, *, bm):
    n, d = h.shape
    grid_rows = n // bm
    c_strips = grid_rows if n <= _RESIDENT_MAX else _C_BIG

    def out_map(p, i):
        return (0, jnp.where(p == 3, i, 0))

    scratch = [
        pltpu.VMEM((_K, bm, n), jnp.float32),
        pltpu.SemaphoreType.DMA((_K,)),
        pltpu.VMEM((n, n), jnp.bfloat16),
        pltpu.VMEM((c_strips * bm, n), jnp.bfloat16),
        pltpu.VMEM((d, n), jnp.float32),
        pltpu.VMEM((d, n), jnp.float32),
        pltpu.VMEM((d, n), jnp.float32),
    ]

    ht = h.T
    outt = pl.pallas_call(
        functools.partial(_prop_body, bm=bm, grid_rows=grid_rows,
                          c_strips=c_strips),
        grid=(4, grid_rows),
        in_specs=[
            pl.BlockSpec(memory_space=pl.ANY),
            pl.BlockSpec(memory_space=pl.ANY),
            pl.BlockSpec((d, n), lambda p, i: (0, 0)),
        ],
        out_specs=pl.BlockSpec((d, bm), out_map),
        out_shape=jax.ShapeDtypeStruct((d, n), jnp.float32),
        scratch_shapes=scratch,
        compiler_params=pltpu.CompilerParams(
            dimension_semantics=("arbitrary", "arbitrary"),
        ),
    )(a2, a1, ht)
    return outt.T


def kernel(adj_u1, adj_u2, adj_i1, adj_i2, adj_m1, adj_m2, adj_a1, adj_a2,
           user_emb, item_emb, mashup_tag_emb, api_tag_emb):
    u = _prop(adj_u1, adj_u2, user_emb, bm=256)
    i = _prop(adj_i1, adj_i2, item_emb, bm=256)
    m = _prop(adj_m1, adj_m2, mashup_tag_emb, bm=512)
    a = _prop(adj_a1, adj_a2, api_tag_emb, bm=512)
    return (u, i, m, a)


# speedup vs baseline: 1.0242x; 1.0141x over previous
import functools

import jax
import jax.numpy as jnp
from jax import lax
from jax.experimental import pallas as pl
from jax.experimental.pallas import tpu as pltpu

_K = 4

_DIMS = (((1,), (1,)), ((), ()))


def _dott(lhs, rhs):
    return lax.dot_general(lhs, rhs, _DIMS, preferred_element_type=jnp.float32)


def _fused_body(a2_0, a1_0, a2_1, a1_1, a2_2, a1_2, a2_3, a1_3,
                ht0, ht1, ht2, ht3, out0, out1, out2, out3,
                buf, sem, cache, t1, e1, t2, *, bm, n_big, n_small):
    s = pl.program_id(0)
    gb = n_big // bm
    gs = n_small // bm

    big = [(0, 0, a2_0, a1_0, ht0, out0), (4 * gb, 3 * gb, a2_1, a1_1, ht1, out1)]
    small_base = 8 * gb
    small_dma = 6 * gb
    small = [(small_base, small_dma, a2_2, a1_2, ht2, out2),
             (small_base + 4 * gs, small_dma + 2 * gs, a2_3, a1_3, ht3, out3)]
    n_pos = 6 * gb + 4 * gs

    segments = []
    for _base, dma0, a2r, a1r, _ht, _out in big:
        segments += [(dma0, gb, a2r, n_big), (dma0 + gb, gb, a1r, n_big),
                     (dma0 + 2 * gb, gb, a1r, n_big)]
    for _base, dma0, a2r, a1r, _ht, _out in small:
        segments += [(dma0, gs, a2r, n_small), (dma0 + gs, gs, a1r, n_small)]

    def issue(pos, slot):
        for start, count, ref, width in segments:
            @pl.when((pos >= start) & (pos < start + count))
            def _(start=start, ref=ref, width=width):
                strip = pos - start
                dst = (buf.at[slot] if width == n_big
                       else buf.at[slot, :, pl.ds(0, width)])
                pltpu.make_async_copy(
                    ref.at[pl.ds(strip * bm, bm), :], dst, sem.at[slot]
                ).start()

    def wait_strip(slot, ref):
        width = ref.shape[1]
        dst = (buf.at[slot] if width == n_big
               else buf.at[slot, :, pl.ds(0, width)])
        pltpu.make_async_copy(ref.at[pl.ds(0, bm), :], dst, sem.at[slot]).wait()

    def big_local_pos(local):
        return jnp.where(local >= 3 * gb, local - gb, local)

    l0 = s
    l1 = s - 4 * gb
    l2 = s - small_base
    l3 = s - (small_base + 4 * gs)
    pos = jnp.where(
        s < 4 * gb, big_local_pos(l0),
        jnp.where(s < 8 * gb, 3 * gb + big_local_pos(l1),
                  jnp.where(s < small_base + 4 * gs, small_dma + l2,
                            small_dma + 2 * gs + l3)))
    big_consum0 = (l0 < 2 * gb) | (l0 >= 3 * gb)
    big_consum1 = (l1 < 2 * gb) | (l1 >= 3 * gb)
    consuming = jnp.where(
        s < 4 * gb, big_consum0,
        jnp.where(s < 8 * gb, big_consum1,
                  jnp.where(s < small_base + 4 * gs, l2 < 2 * gs, l3 < 2 * gs)))
    slot = lax.rem(pos, _K)

    @pl.when(s == 0)
    def _prologue():
        for k in range(_K):
            issue(jnp.int32(k), jnp.int32(k))

    def big_graph(base, a2r, a1r, ht, out):
        local = s - base
        p = local // gb
        i = lax.rem(local, gb)
        rows = pl.ds(i * bm, bm)

        @pl.when(p == 0)
        def _p0():
            wait_strip(slot, a2r)
            blk = buf[slot].astype(jnp.bfloat16)
            cache[rows, :] = blk
            t1[:, rows] = _dott(ht[...].astype(jnp.bfloat16), blk)

        @pl.when(p == 1)
        def _p1():
            wait_strip(slot, a1r)
            e1[:, rows] = _dott(t1[...].astype(jnp.bfloat16),
                                buf[slot].astype(jnp.bfloat16))

        @pl.when(p == 2)
        def _p2():
            t2[:, rows] = _dott(e1[...].astype(jnp.bfloat16), cache[rows, :])

        @pl.when(p == 3)
        def _p3():
            wait_strip(slot, a1r)
            e2_blk = _dott(t2[...].astype(jnp.bfloat16),
                           buf[slot].astype(jnp.bfloat16))
            out[...] = (ht[:, rows] + e1[:, rows] + e2_blk) * (1.0 / 3.0)

    def small_graph(base, a2r, a1r, ht, out):
        local = s - base
        p = local // gs
        i = lax.rem(local, gs)
        rows = pl.ds(i * bm, bm)
        ncols = pl.ds(0, n_small)

        @pl.when(p == 0)
        def _p0():
            wait_strip(slot, a2r)
            blk = buf[slot, :, ncols].astype(jnp.bfloat16)
            cache[rows, ncols] = blk
            t1[:, rows] = _dott(ht[...].astype(jnp.bfloat16), blk)

        @pl.when(p == 1)
        def _p1():
            wait_strip(slot, a1r)
            blk = buf[slot, :, ncols].astype(jnp.bfloat16)
            cache[pl.ds(n_small + i * bm, bm), ncols] = blk
            e1[:, rows] = _dott(t1[:, ncols].astype(jnp.bfloat16), blk)

        @pl.when(p == 2)
        def _p2():
            t2[:, rows] = _dott(e1[:, ncols].astype(jnp.bfloat16),
                                cache[rows, ncols])

        @pl.when(p == 3)
        def _p3():
            e2_blk = _dott(t2[:, ncols].astype(jnp.bfloat16),
                           cache[pl.ds(n_small + i * bm, bm), ncols])
            out[...] = (ht[:, rows] + e1[:, rows] + e2_blk) * (1.0 / 3.0)

    for base, _dma0, a2r, a1r, ht, out in big:
        @pl.when((s >= base) & (s < base + 4 * gb))
        def _(base=base, a2r=a2r, a1r=a1r, ht=ht, out=out):
            big_graph(base, a2r, a1r, ht, out)

    for base, _dma0, a2r, a1r, ht, out in small:
        @pl.when((s >= base) & (s < base + 4 * gs))
        def _(base=base, a2r=a2r, a1r=a1r, ht=ht, out=out):
            small_graph(base, a2r, a1r, ht, out)

    nxt = pos + _K

    @pl.when(consuming & (nxt < n_pos))
    def _refill():
        issue(nxt, slot)


def _run(adjs, hts, *, bm):
    n_big = adjs[0].shape[0]
    n_small = adjs[4].shape[0]
    d = hts[0].shape[0]
    gb, gs = n_big // bm, n_small // bm
    steps = 8 * gb + 8 * gs

    def ht_spec(n):
        return pl.BlockSpec((d, n), lambda s: (0, 0))

    def out_spec(base, g):
        return pl.BlockSpec(
            (d, bm), lambda s, base=base, g=g: (0, jnp.clip(s - base - 3 * g,
                                                            0, g - 1)))

    out_specs = (out_spec(0, gb), out_spec(4 * gb, gb),
                 out_spec(8 * gb, gs), out_spec(8 * gb + 4 * gs, gs))
    out_shape = tuple(jax.ShapeDtypeStruct((d, n), jnp.float32)
                      for n in (n_big, n_big, n_small, n_small))

    scratch = [
        pltpu.VMEM((_K, bm, n_big), jnp.float32),
        pltpu.SemaphoreType.DMA((_K,)),
        pltpu.VMEM((n_big, n_big), jnp.bfloat16),
        pltpu.VMEM((d, n_big), jnp.float32),
        pltpu.VMEM((d, n_big), jnp.float32),
        pltpu.VMEM((d, n_big), jnp.float32),
    ]

    return pl.pallas_call(
        functools.partial(_fused_body, bm=bm, n_big=n_big, n_small=n_small),
        grid=(steps,),
        in_specs=[pl.BlockSpec(memory_space=pl.ANY)] * 8 + [
            ht_spec(n_big), ht_spec(n_big), ht_spec(n_small), ht_spec(n_small)],
        out_specs=out_specs,
        out_shape=out_shape,
        scratch_shapes=scratch,
        compiler_params=pltpu.CompilerParams(
            dimension_semantics=("arbitrary",),
        ),
    )(*adjs, *hts)


def kernel(adj_u1, adj_u2, adj_i1, adj_i2, adj_m1, adj_m2, adj_a1, adj_a2,
           user_emb, item_emb, mashup_tag_emb, api_tag_emb):
    adjs = (adj_u2, adj_u1, adj_i2, adj_i1, adj_m2, adj_m1, adj_a2, adj_a1)
    hts = (user_emb.T, item_emb.T, mashup_tag_emb.T, api_tag_emb.T)
    u, i, m, a = _run(adjs, hts, bm=256)
    return (u.T, i.T, m.T, a.T)
